# Initial kernel scaffold; baseline (speedup 1.0000x reference)
import jax
import jax.numpy as jnp
from jax.experimental import pallas as pl


def kernel(x, edge_index, edge_attr, We1, be1, W1a, b1a, W1b, b1b, We2, be2, W2a, b2a, W2b, b2b, Wm1, bm1, Wm2, bm2):
    raise NotImplementedError("write your pallas kernel here")


# speedup vs baseline: 2.2035x; 2.2035x over previous
import functools

import jax
import jax.numpy as jnp
from jax import lax
from jax.experimental import pallas as pl
from jax.experimental.pallas import tpu as pltpu
from jax.experimental.pallas import tpu_sc as plsc

N = 10000
E = 320000
D = 128
H = 64

NC = 2
NS = 16
L = 16
CH = 80
EPC = E // NS
EPW = E // (NC * NS)
NP = 10240
NPT = NP // NS

_mesh = plsc.VectorSubcoreMesh(core_axis_name="c", subcore_axis_name="s")


def _make_sc_aggregate(W):
    KV = W // L
    n_chunks = EPC // CH

    @functools.partial(
        pl.kernel,
        out_type=jax.ShapeDtypeStruct((2 * NP, W), jnp.float32),
        mesh=_mesh,
        compiler_params=pltpu.CompilerParams(use_tc_tiling_on_sc=False),
        scratch_types=[
            pltpu.VMEM((CH,), jnp.int32),
            pltpu.VMEM((CH,), jnp.int32),
            pltpu.VMEM((2 * CH,), jnp.float32),
            pltpu.VMEM((CH, W), jnp.float32),
            pltpu.VMEM((3 * W,), jnp.float32),
            pltpu.VMEM_SHARED((NP, W), jnp.float32),
            pltpu.SemaphoreType.DMA,
        ],
    )
    def k(xg, srcv, dstv, ea2, wec, zeros, out, sidx, didx, eav, grow, wbuf, aggr, sem):
        c = lax.axis_index("c")
        s = lax.axis_index("s")

        pltpu.sync_copy(zeros.at[pl.ds(s * NPT, NPT)], aggr.at[pl.ds(s * NPT, NPT)])
        pltpu.sync_copy(wec.at[c], wbuf)
        w0 = [wbuf[pl.ds(L * k2, L)] for k2 in range(KV)]
        w1 = [wbuf[pl.ds(W + L * k2, L)] for k2 in range(KV)]
        bv = [wbuf[pl.ds(2 * W + L * k2, L)] for k2 in range(KV)]
        plsc.subcore_barrier()

        cN = c * N
        ebase = s * EPC

        def chunk_body(kk, _):
            base = ebase + kk * CH
            pltpu.sync_copy(srcv.at[pl.ds(base, CH)], sidx)
            pltpu.sync_copy(dstv.at[pl.ds(base, CH)], didx)
            pltpu.sync_copy(ea2.at[pl.ds(2 * base, 2 * CH)], eav)
            for k2 in range(CH // L):
                sidx[pl.ds(L * k2, L)] = sidx[pl.ds(L * k2, L)] + cN
            pltpu.async_copy(xg.at[sidx], grow, sem).wait()
            def group_body(jj, _2):
                ea_a = eav[pl.ds(2 * L * jj, L)]
                ea_b = eav[pl.ds(2 * L * jj + L, L)]
                for i in range(L):
                    j = jj * L + i
                    pv = ea_a if i < 8 else ea_b
                    e0 = jnp.full((L,), pv[(2 * i) % L])
                    e1 = jnp.full((L,), pv[(2 * i + 1) % L])
                    for k2 in range(KV):
                        g = grow[j, pl.ds(L * k2, L)]
                        m = g + e0 * w0[k2] + e1 * w1[k2] + bv[k2]
                        grow[j, pl.ds(L * k2, L)] = jnp.maximum(m, 0.0)
                return 0
            lax.fori_loop(0, CH // L, group_body, 0)
            pltpu.sync_copy(grow, aggr.at[didx], add=True)
            return 0

        lax.fori_loop(0, n_chunks, chunk_body, 0)
        plsc.subcore_barrier()
        pltpu.sync_copy(aggr.at[pl.ds(s * NPT, NPT)],
                        out.at[pl.ds(c * NP + s * NPT, NPT)])

    return k


_sc_aggr64 = _make_sc_aggregate(64)
_sc_aggr32 = _make_sc_aggregate(32)


def _make_sc_edge_mlp():
    KV = H // L
    n_chunks = EPW // CH

    @functools.partial(
        pl.kernel,
        out_type=jax.ShapeDtypeStruct((E,), jnp.float32),
        mesh=_mesh,
        compiler_params=pltpu.CompilerParams(use_tc_tiling_on_sc=False),
        scratch_types=[
            pltpu.VMEM((CH,), jnp.int32),
            pltpu.VMEM((CH,), jnp.int32),
            pltpu.VMEM((2 * CH,), jnp.float32),
            pltpu.VMEM((CH, H), jnp.float32),
            pltpu.VMEM((CH, H), jnp.float32),
            pltpu.VMEM((CH,), jnp.float32),
            pltpu.VMEM((208,), jnp.float32),
            pltpu.SemaphoreType.DMA,
            pltpu.SemaphoreType.DMA,
        ],
    )
    def k(ta, tb, srcv, dstv, ea2, w3, out, sidx, didx, eav, ga, gb, lbuf, wbuf, sema, semb):
        c = lax.axis_index("c")
        s = lax.axis_index("s")
        wid = s * NC + c
        ebase = wid * EPW

        pltpu.sync_copy(w3, wbuf)
        w0 = [wbuf[pl.ds(L * k2, L)] for k2 in range(KV)]
        w1 = [wbuf[pl.ds(H + L * k2, L)] for k2 in range(KV)]
        wm = [wbuf[pl.ds(2 * H + L * k2, L)] for k2 in range(KV)]
        b2v = wbuf[pl.ds(3 * H, L)]

        def chunk_body(kk, _):
            base = ebase + kk * CH
            pltpu.sync_copy(srcv.at[pl.ds(base, CH)], sidx)
            pltpu.sync_copy(dstv.at[pl.ds(base, CH)], didx)
            pltpu.sync_copy(ea2.at[pl.ds(2 * base, 2 * CH)], eav)
            cpa = pltpu.async_copy(ta.at[sidx], ga, sema)
            cpb = pltpu.async_copy(tb.at[didx], gb, semb)
            cpa.wait()
            cpb.wait()

            lane = lax.iota(jnp.int32, L)

            def group_body(jj, _2):
                ea_a = eav[pl.ds(2 * L * jj, L)]
                ea_b = eav[pl.ds(2 * L * jj + L, L)]
                outv = jnp.zeros((L,), jnp.float32)
                for i in range(L):
                    j = jj * L + i
                    pv = ea_a if i < 8 else ea_b
                    e0 = jnp.full((L,), pv[(2 * i) % L])
                    e1 = jnp.full((L,), pv[(2 * i + 1) % L])
                    acc = b2v
                    for k2 in range(KV):
                        z = ga[j, pl.ds(L * k2, L)] + gb[j, pl.ds(L * k2, L)]
                        z = jnp.maximum(z + e0 * w0[k2] + e1 * w1[k2], 0.0)
                        acc = acc + z * wm[k2]
                    for sh in (1, 2, 4, 8):
                        acc = acc + acc.at[lane ^ sh].get(mode="promise_in_bounds")
                    outv = jnp.where(lane == i, acc, outv)
                lbuf[pl.ds(jj * L, L)] = outv
                return 0
            lax.fori_loop(0, CH // L, group_body, 0)
            pltpu.sync_copy(lbuf, out.at[pl.ds(base, CH)])
            return 0

        lax.fori_loop(0, n_chunks, chunk_body, 0)

    return k


_sc_edge_mlp = _make_sc_edge_mlp()


_TC_ROWS = 400
_TC_GRID = N // _TC_ROWS


def _tc1_body(x_ref, aa_ref, ab_ref, w1a_ref, b1a_ref, w1b_ref, b1b_ref, be2_ref,
              h_ref, hs_ref):
    g = x_ref[...] + jnp.concatenate([aa_ref[...], ab_ref[...]], axis=1)
    t = jnp.maximum(jnp.dot(g, w1a_ref[...], preferred_element_type=jnp.float32)
                    + b1a_ref[...], 0.0)
    u = jnp.dot(t, w1b_ref[...], preferred_element_type=jnp.float32) + b1b_ref[...]
    h = jnp.maximum(u, 0.0)
    h_ref[...] = h
    hs_ref[...] = h + be2_ref[...]


def _tc2_body(h_ref, aa_ref, ab_ref, w2a_ref, b2a_ref, w2b_ref, b2b_ref,
              wms_ref, wmd_ref, bm1_ref, a_ref, b_ref):
    g = h_ref[...] + jnp.concatenate([aa_ref[...], ab_ref[...]], axis=1)
    t = jnp.maximum(jnp.dot(g, w2a_ref[...], preferred_element_type=jnp.float32)
                    + b2a_ref[...], 0.0)
    h2 = jnp.dot(t, w2b_ref[...], preferred_element_type=jnp.float32) + b2b_ref[...]
    a_ref[...] = jnp.dot(h2, wms_ref[...], preferred_element_type=jnp.float32) + bm1_ref[...]
    b_ref[...] = jnp.dot(h2, wmd_ref[...], preferred_element_type=jnp.float32)


def _row_spec(w):
    return pl.BlockSpec((_TC_ROWS, w), lambda i: (i, 0))


def _full_spec(shape):
    return pl.BlockSpec(shape, lambda i: tuple(0 for _ in shape))


_tc1 = pl.pallas_call(
    _tc1_body,
    grid=(_TC_GRID,),
    in_specs=[
        _row_spec(D), _row_spec(64), _row_spec(64),
        _full_spec((D, H)), _full_spec((1, H)),
        _full_spec((H, H)), _full_spec((1, H)),
        _full_spec((1, H)),
    ],
    out_specs=[_row_spec(H), _row_spec(H)],
    out_shape=[jax.ShapeDtypeStruct((N, H), jnp.float32),
               jax.ShapeDtypeStruct((N, H), jnp.float32)],
)

_tc2 = pl.pallas_call(
    _tc2_body,
    grid=(_TC_GRID,),
    in_specs=[
        _row_spec(H), _row_spec(32), _row_spec(32),
        _full_spec((H, H)), _full_spec((1, H)),
        _full_spec((H, H)), _full_spec((1, H)),
        _full_spec((H, H)), _full_spec((H, H)), _full_spec((1, H)),
    ],
    out_specs=[_row_spec(H), _row_spec(H)],
    out_shape=[jax.ShapeDtypeStruct((N, H), jnp.float32),
               jax.ShapeDtypeStruct((N, H), jnp.float32)],
)


def kernel(x, edge_index, edge_attr, We1, be1, W1a, b1a, W1b, b1b,
           We2, be2, W2a, b2a, W2b, b2b, Wm1, bm1, Wm2, bm2):
    src = edge_index[0]
    dst = edge_index[1]
    ea2 = edge_attr.reshape(-1)

    xg = jnp.concatenate([x[:, :64], x[:, 64:]], axis=0)
    wec1 = jnp.stack([
        jnp.concatenate([We1[0, :64], We1[1, :64], be1[:64]]),
        jnp.concatenate([We1[0, 64:], We1[1, 64:], be1[64:]]),
    ])
    z64 = jnp.zeros((NP, 64), jnp.float32)
    out1 = _sc_aggr64(xg, src, dst, ea2, wec1, z64)

    h, hs = _tc1(x, out1[:N], out1[NP:NP + N],
                 W1a, b1a.reshape(1, H), W1b, b1b.reshape(1, H),
                 be2.reshape(1, H))

    hg = jnp.concatenate([hs[:, :32], hs[:, 32:]], axis=0)
    wec2 = jnp.stack([
        jnp.concatenate([We2[0, :32], We2[1, :32], jnp.zeros((32,), jnp.float32)]),
        jnp.concatenate([We2[0, 32:], We2[1, 32:], jnp.zeros((32,), jnp.float32)]),
    ])
    z32 = jnp.zeros((NP, 32), jnp.float32)
    out2 = _sc_aggr32(hg, src, dst, ea2, wec2, z32)

    A, B = _tc2(h, out2[:N], out2[NP:NP + N],
                W2a, b2a.reshape(1, H), W2b, b2b.reshape(1, H),
                Wm1[:H], Wm1[H:2 * H], bm1.reshape(1, H))

    w3 = jnp.concatenate([Wm1[2 * H], Wm1[2 * H + 1], Wm2[:, 0],
                          jnp.concatenate([bm2, jnp.zeros((15,), jnp.float32)])])
    logits = _sc_edge_mlp(A, B, src, dst, ea2, w3)
    return logits


# speedup vs baseline: 5.3137x; 2.4115x over previous
import functools

import jax
import jax.numpy as jnp
from jax import lax
from jax.experimental import pallas as pl
from jax.experimental.pallas import tpu as pltpu
from jax.experimental.pallas import tpu_sc as plsc

N = 10000
E = 320000
D = 128
H = 64

NC = 2
NS = 16
L = 16
SUB = 80
CH = 400
NSUB = CH // SUB
EPC = E // NS
EPW = E // (NC * NS)
NP = 10240
NPT = NP // NS

_mesh = plsc.VectorSubcoreMesh(core_axis_name="c", subcore_axis_name="s")
_params = pltpu.CompilerParams(use_tc_tiling_on_sc=False)


def _make_sc_aggregate(W):
    KV = W // L
    n_chunks = EPC // CH

    @functools.partial(
        pl.kernel,
        out_type=jax.ShapeDtypeStruct((2 * NP, W), jnp.float32),
        mesh=_mesh,
        compiler_params=_params,
        scratch_types=[
            pltpu.VMEM((4, CH), jnp.int32),
            pltpu.VMEM((4, NSUB, SUB), jnp.int32),
            pltpu.VMEM((4, 2 * CH), jnp.float32),
            pltpu.VMEM((2, CH, W), jnp.float32),
            pltpu.VMEM((3 * W,), jnp.float32),
            pltpu.VMEM_SHARED((NP, W), jnp.float32),
            pltpu.SemaphoreType.DMA,
            pltpu.SemaphoreType.DMA,
            pltpu.SemaphoreType.DMA,
        ],
    )
    def k(xga, xgb, srcv, dstv, ea2, wec, zeros, out,
          fsrc, fdst, feav, grow, wbuf, aggr, isem, gsem, ssem):
        c = lax.axis_index("c")
        s = lax.axis_index("s")

        pltpu.sync_copy(zeros.at[pl.ds(s * NPT, NPT)], aggr.at[pl.ds(s * NPT, NPT)])
        pltpu.sync_copy(wec.at[c], wbuf)
        w0 = [wbuf[pl.ds(L * k2, L)] for k2 in range(KV)]
        w1 = [wbuf[pl.ds(W + L * k2, L)] for k2 in range(KV)]
        bv = [wbuf[pl.ds(2 * W + L * k2, L)] for k2 in range(KV)]
        plsc.subcore_barrier()

        ebase = s * EPC

        def fire_fetch(kk):
            b4 = lax.rem(kk, 4)
            base = ebase + kk * CH
            pltpu.async_copy(srcv.at[pl.ds(base, CH)], fsrc.at[b4], isem)
            for i in range(NSUB):
                pltpu.async_copy(dstv.at[pl.ds(base + i * SUB, SUB)],
                                 fdst.at[b4, i], isem)
            pltpu.async_copy(ea2.at[pl.ds(2 * base, 2 * CH)], feav.at[b4], isem)

        def wait_fetch():
            pltpu.make_async_copy(srcv.at[pl.ds(0, CH)], fsrc.at[0], isem).wait()
            for i in range(NSUB):
                pltpu.make_async_copy(dstv.at[pl.ds(0, SUB)], fdst.at[0, i], isem).wait()
            pltpu.make_async_copy(ea2.at[pl.ds(0, 2 * CH)], feav.at[0], isem).wait()

        def fire_gather(kk):
            b4 = lax.rem(kk, 4)
            b2 = lax.rem(kk, 2)

            @pl.when(c == 0)
            def _():
                for i in range(NSUB):
                    pltpu.async_copy(xga.at[fsrc.at[b4, pl.ds(i * SUB, SUB)]],
                                     grow.at[b2, pl.ds(i * SUB, SUB)], gsem)

            @pl.when(c == 1)
            def _():
                for i in range(NSUB):
                    pltpu.async_copy(xgb.at[fsrc.at[b4, pl.ds(i * SUB, SUB)]],
                                     grow.at[b2, pl.ds(i * SUB, SUB)], gsem)

        def wait_gather():
            for i in range(NSUB):
                pltpu.make_async_copy(xga.at[fsrc.at[0, pl.ds(0, SUB)]],
                                      grow.at[0, pl.ds(0, SUB)], gsem).wait()

        def fire_scatter(kk):
            b4 = lax.rem(kk, 4)
            b2 = lax.rem(kk, 2)
            for i in range(NSUB):
                pltpu.async_copy(grow.at[b2, pl.ds(i * SUB, SUB)],
                                 aggr.at[fdst.at[b4, i]], ssem, add=True)

        def wait_scatter():
            for i in range(NSUB):
                pltpu.make_async_copy(grow.at[0, pl.ds(0, SUB)],
                                      aggr.at[fdst.at[0, 0]], ssem).wait()

        def compute(kk):
            b4 = lax.rem(kk, 4)
            b2 = lax.rem(kk, 2)

            def group_body(jj, _2):
                ea_a = feav[b4, pl.ds(2 * L * jj, L)]
                ea_b = feav[b4, pl.ds(2 * L * jj + L, L)]
                for i in range(L):
                    j = jj * L + i
                    pv = ea_a if i < 8 else ea_b
                    e0 = jnp.full((L,), pv[(2 * i) % L])
                    e1 = jnp.full((L,), pv[(2 * i + 1) % L])
                    for k2 in range(KV):
                        g = grow[b2, j, pl.ds(L * k2, L)]
                        m = g + e0 * w0[k2] + e1 * w1[k2] + bv[k2]
                        grow[b2, j, pl.ds(L * k2, L)] = jnp.maximum(m, 0.0)
                return 0
            lax.fori_loop(0, CH // L, group_body, 0)

        fire_fetch(0)
        wait_fetch()
        fire_gather(0)
        fire_fetch(1)

        def body(kk, _):
            wait_gather()
            compute(kk)

            @pl.when(kk < n_chunks - 1)
            def _():
                wait_fetch()

            @pl.when(kk > 0)
            def _():
                wait_scatter()

            @pl.when(kk < n_chunks - 1)
            def _():
                fire_gather(kk + 1)

            fire_scatter(kk)

            @pl.when(kk < n_chunks - 2)
            def _():
                fire_fetch(kk + 2)
            return 0

        lax.fori_loop(0, n_chunks, body, 0)
        wait_scatter()
        plsc.subcore_barrier()
        pltpu.sync_copy(aggr.at[pl.ds(s * NPT, NPT)],
                        out.at[pl.ds(c * NP + s * NPT, NPT)])

    return k


_sc_aggr64 = _make_sc_aggregate(64)
_sc_aggr32 = _make_sc_aggregate(32)


def _make_sc_edge_mlp():
    KV = H // L
    n_chunks = EPW // CH

    @functools.partial(
        pl.kernel,
        out_type=jax.ShapeDtypeStruct((E,), jnp.float32),
        mesh=_mesh,
        compiler_params=_params,
        scratch_types=[
            pltpu.VMEM((4, CH), jnp.int32),
            pltpu.VMEM((4, CH), jnp.int32),
            pltpu.VMEM((4, 2 * CH), jnp.float32),
            pltpu.VMEM((2, CH, H), jnp.float32),
            pltpu.VMEM((2, CH, H), jnp.float32),
            pltpu.VMEM((2, CH), jnp.float32),
            pltpu.VMEM((208,), jnp.float32),
            pltpu.SemaphoreType.DMA,
            pltpu.SemaphoreType.DMA,
            pltpu.SemaphoreType.DMA,
        ],
    )
    def k(ta, tb, srcv, dstv, ea2, w3, out,
          fsrc, fdst, feav, ga, gb, lbuf, wbuf, isem, gsem, osem):
        c = lax.axis_index("c")
        s = lax.axis_index("s")
        wid = s * NC + c
        ebase = wid * EPW

        pltpu.sync_copy(w3, wbuf)
        w0 = [wbuf[pl.ds(L * k2, L)] for k2 in range(KV)]
        w1 = [wbuf[pl.ds(H + L * k2, L)] for k2 in range(KV)]
        wm = [wbuf[pl.ds(2 * H + L * k2, L)] for k2 in range(KV)]
        b2v = wbuf[pl.ds(3 * H, L)]
        lane = lax.iota(jnp.int32, L)

        def fire_fetch(kk):
            b4 = lax.rem(kk, 4)
            base = ebase + kk * CH
            pltpu.async_copy(srcv.at[pl.ds(base, CH)], fsrc.at[b4], isem)
            pltpu.async_copy(dstv.at[pl.ds(base, CH)], fdst.at[b4], isem)
            pltpu.async_copy(ea2.at[pl.ds(2 * base, 2 * CH)], feav.at[b4], isem)

        def wait_fetch():
            pltpu.make_async_copy(srcv.at[pl.ds(0, CH)], fsrc.at[0], isem).wait()
            pltpu.make_async_copy(dstv.at[pl.ds(0, CH)], fdst.at[0], isem).wait()
            pltpu.make_async_copy(ea2.at[pl.ds(0, 2 * CH)], feav.at[0], isem).wait()

        def fire_gather(kk):
            b4 = lax.rem(kk, 4)
            b2 = lax.rem(kk, 2)
            for i in range(NSUB):
                pltpu.async_copy(ta.at[fsrc.at[b4, pl.ds(i * SUB, SUB)]],
                                 ga.at[b2, pl.ds(i * SUB, SUB)], gsem)
                pltpu.async_copy(tb.at[fdst.at[b4, pl.ds(i * SUB, SUB)]],
                                 gb.at[b2, pl.ds(i * SUB, SUB)], gsem)

        def wait_gather():
            for i in range(NSUB):
                pltpu.make_async_copy(ta.at[fsrc.at[0, pl.ds(0, SUB)]],
                                      ga.at[0, pl.ds(0, SUB)], gsem).wait()
                pltpu.make_async_copy(tb.at[fdst.at[0, pl.ds(0, SUB)]],
                                      gb.at[0, pl.ds(0, SUB)], gsem).wait()

        def compute(kk):
            b4 = lax.rem(kk, 4)
            b2 = lax.rem(kk, 2)

            def group_body(jj, _2):
                ea_a = feav[b4, pl.ds(2 * L * jj, L)]
                ea_b = feav[b4, pl.ds(2 * L * jj + L, L)]
                outv = jnp.zeros((L,), jnp.float32)
                for i in range(L):
                    j = jj * L + i
                    pv = ea_a if i < 8 else ea_b
                    e0 = jnp.full((L,), pv[(2 * i) % L])
                    e1 = jnp.full((L,), pv[(2 * i + 1) % L])
                    acc = b2v
                    for k2 in range(KV):
                        z = ga[b2, j, pl.ds(L * k2, L)] + gb[b2, j, pl.ds(L * k2, L)]
                        z = jnp.maximum(z + e0 * w0[k2] + e1 * w1[k2], 0.0)
                        acc = acc + z * wm[k2]
                    for sh in (1, 2, 4, 8):
                        acc = acc + acc.at[lane ^ sh].get(mode="promise_in_bounds")
                    outv = jnp.where(lane == i, acc, outv)
                lbuf[b2, pl.ds(jj * L, L)] = outv
                return 0
            lax.fori_loop(0, CH // L, group_body, 0)

        fire_fetch(0)
        wait_fetch()
        fire_gather(0)
        fire_fetch(1)

        def body(kk, _):
            wait_gather()
            compute(kk)

            @pl.when(kk < n_chunks - 1)
            def _():
                wait_fetch()
                fire_gather(kk + 1)

            @pl.when(kk > 0)
            def _():
                pltpu.make_async_copy(lbuf.at[0], out.at[pl.ds(0, CH)], osem).wait()

            b2 = lax.rem(kk, 2)
            pltpu.async_copy(lbuf.at[b2], out.at[pl.ds(ebase + kk * CH, CH)], osem)

            @pl.when(kk < n_chunks - 2)
            def _():
                fire_fetch(kk + 2)
            return 0

        lax.fori_loop(0, n_chunks, body, 0)
        pltpu.make_async_copy(lbuf.at[0], out.at[pl.ds(0, CH)], osem).wait()

    return k


_sc_edge_mlp = _make_sc_edge_mlp()


_TC_ROWS = 400
_TC_GRID = N // _TC_ROWS


def _tc1_body(x_ref, aa_ref, ab_ref, w1a_ref, b1a_ref, w1b_ref, b1b_ref, be2_ref,
              h_ref, hs_ref):
    g = x_ref[...] + jnp.concatenate([aa_ref[...], ab_ref[...]], axis=1)
    t = jnp.maximum(jnp.dot(g, w1a_ref[...], preferred_element_type=jnp.float32)
                    + b1a_ref[...], 0.0)
    u = jnp.dot(t, w1b_ref[...], preferred_element_type=jnp.float32) + b1b_ref[...]
    h = jnp.maximum(u, 0.0)
    h_ref[...] = h
    hs_ref[...] = h + be2_ref[...]


def _tc2_body(h_ref, aa_ref, ab_ref, w2a_ref, b2a_ref, w2b_ref, b2b_ref,
              wms_ref, wmd_ref, bm1_ref, a_ref, b_ref):
    g = h_ref[...] + jnp.concatenate([aa_ref[...], ab_ref[...]], axis=1)
    t = jnp.maximum(jnp.dot(g, w2a_ref[...], preferred_element_type=jnp.float32)
                    + b2a_ref[...], 0.0)
    h2 = jnp.dot(t, w2b_ref[...], preferred_element_type=jnp.float32) + b2b_ref[...]
    a_ref[...] = jnp.dot(h2, wms_ref[...], preferred_element_type=jnp.float32) + bm1_ref[...]
    b_ref[...] = jnp.dot(h2, wmd_ref[...], preferred_element_type=jnp.float32)


def _row_spec(w):
    return pl.BlockSpec((_TC_ROWS, w), lambda i: (i, 0))


def _full_spec(shape):
    return pl.BlockSpec(shape, lambda i: tuple(0 for _ in shape))


_tc1 = pl.pallas_call(
    _tc1_body,
    grid=(_TC_GRID,),
    in_specs=[
        _row_spec(D), _row_spec(64), _row_spec(64),
        _full_spec((D, H)), _full_spec((1, H)),
        _full_spec((H, H)), _full_spec((1, H)),
        _full_spec((1, H)),
    ],
    out_specs=[_row_spec(H), _row_spec(H)],
    out_shape=[jax.ShapeDtypeStruct((N, H), jnp.float32),
               jax.ShapeDtypeStruct((N, H), jnp.float32)],
)

_tc2 = pl.pallas_call(
    _tc2_body,
    grid=(_TC_GRID,),
    in_specs=[
        _row_spec(H), _row_spec(32), _row_spec(32),
        _full_spec((H, H)), _full_spec((1, H)),
        _full_spec((H, H)), _full_spec((1, H)),
        _full_spec((H, H)), _full_spec((H, H)), _full_spec((1, H)),
    ],
    out_specs=[_row_spec(H), _row_spec(H)],
    out_shape=[jax.ShapeDtypeStruct((N, H), jnp.float32),
               jax.ShapeDtypeStruct((N, H), jnp.float32)],
)


def kernel(x, edge_index, edge_attr, We1, be1, W1a, b1a, W1b, b1b,
           We2, be2, W2a, b2a, W2b, b2b, Wm1, bm1, Wm2, bm2):
    src = edge_index[0]
    dst = edge_index[1]
    ea2 = edge_attr.reshape(-1)

    xga = x[:, :64]
    xgb = x[:, 64:]
    wec1 = jnp.stack([
        jnp.concatenate([We1[0, :64], We1[1, :64], be1[:64]]),
        jnp.concatenate([We1[0, 64:], We1[1, 64:], be1[64:]]),
    ])
    z64 = jnp.zeros((NP, 64), jnp.float32)
    out1 = _sc_aggr64(xga, xgb, src, dst, ea2, wec1, z64)

    h, hs = _tc1(x, out1[:N], out1[NP:NP + N],
                 W1a, b1a.reshape(1, H), W1b, b1b.reshape(1, H),
                 be2.reshape(1, H))

    hga = hs[:, :32]
    hgb = hs[:, 32:]
    wec2 = jnp.stack([
        jnp.concatenate([We2[0, :32], We2[1, :32], jnp.zeros((32,), jnp.float32)]),
        jnp.concatenate([We2[0, 32:], We2[1, 32:], jnp.zeros((32,), jnp.float32)]),
    ])
    z32 = jnp.zeros((NP, 32), jnp.float32)
    out2 = _sc_aggr32(hga, hgb, src, dst, ea2, wec2, z32)

    A, B = _tc2(h, out2[:N], out2[NP:NP + N],
                W2a, b2a.reshape(1, H), W2b, b2b.reshape(1, H),
                Wm1[:H], Wm1[H:2 * H], bm1.reshape(1, H))

    w3 = jnp.concatenate([Wm1[2 * H], Wm1[2 * H + 1], Wm2[:, 0],
                          jnp.concatenate([bm2, jnp.zeros((15,), jnp.float32)])])
    logits = _sc_edge_mlp(A, B, src, dst, ea2, w3)
    return logits
